# baseline (device time: 196596 ns/iter reference)
import jax
import jax.numpy as jnp
from jax import lax
from jax.experimental import pallas as pl
from jax.experimental.pallas import tpu as pltpu

N_DEV = 4
WINDOW = 128
HEAD_DIM = 128
SCALE = 0.08838834764831843


def _allreduce_body(p_ref, out_ref, comm_ref, send_sems, recv_sems):
    my = lax.axis_index("i")
    left = lax.rem(my + N_DEV - 1, N_DEV)
    right = lax.rem(my + 1, N_DEV)

    barrier_sem = pltpu.get_barrier_semaphore()
    for nbr in (left, right):
        pl.semaphore_signal(
            barrier_sem, inc=1,
            device_id=(nbr,), device_id_type=pl.DeviceIdType.MESH,
        )
    pl.semaphore_wait(barrier_sem, 2)

    out_ref[...] = p_ref[...]
    comm_ref[0, :, :] = p_ref[...]

    for h in range(N_DEV - 1):
        send_slot = h % 2
        recv_slot = (h + 1) % 2
        rdma = pltpu.make_async_remote_copy(
            src_ref=comm_ref.at[send_slot],
            dst_ref=comm_ref.at[recv_slot],
            send_sem=send_sems.at[send_slot],
            recv_sem=recv_sems.at[recv_slot],
            device_id=(right,),
            device_id_type=pl.DeviceIdType.MESH,
        )
        rdma.start()
        rdma.wait()
        out_ref[...] += comm_ref[recv_slot, :, :]


def _ring_allreduce(partial):
    m, n = partial.shape
    return pl.pallas_call(
        _allreduce_body,
        out_shape=jax.ShapeDtypeStruct((m, n), partial.dtype),
        in_specs=[pl.BlockSpec(memory_space=pltpu.VMEM)],
        out_specs=pl.BlockSpec(memory_space=pltpu.VMEM),
        scratch_shapes=[
            pltpu.VMEM((2, m, n), partial.dtype),
            pltpu.SemaphoreType.DMA((2,)),
            pltpu.SemaphoreType.DMA((2,)),
        ],
        compiler_params=pltpu.CompilerParams(collective_id=0),
    )(partial)


def kernel(x, Wq, K_ext, V_ext, Wo):
    _, Sq, _ = x.shape
    h_loc = Wq.shape[1] // HEAD_DIM
    my = lax.axis_index("i")

    xb = x[0].astype(jnp.bfloat16)
    Q = jnp.dot(xb, Wq.astype(jnp.bfloat16),
                preferred_element_type=jnp.float32)
    Q = Q.reshape(Sq, h_loc, HEAD_DIM)

    K = lax.dynamic_slice_in_dim(K_ext[0], my * h_loc, h_loc, axis=1)
    V = lax.dynamic_slice_in_dim(V_ext[0], my * h_loc, h_loc, axis=1)
    skv = K.shape[0]

    scores = jnp.einsum(
        "ihd,jhd->hij", Q.astype(jnp.bfloat16), K.astype(jnp.bfloat16),
        preferred_element_type=jnp.float32,
    ) * SCALE
    qi = lax.broadcasted_iota(jnp.int32, (Sq, skv), 0)
    ki = lax.broadcasted_iota(jnp.int32, (Sq, skv), 1)
    mask = jnp.abs(qi - ki) <= WINDOW
    scores = jnp.where(mask[None], scores, -1e9)
    scores_max = scores.max(axis=-1, keepdims=True)
    w = jnp.exp(scores - scores_max)
    w = w / w.sum(axis=-1, keepdims=True)

    ctx = jnp.einsum(
        "hij,jhd->ihd", w.astype(jnp.bfloat16), V.astype(jnp.bfloat16),
        preferred_element_type=jnp.float32,
    ).reshape(Sq, h_loc * HEAD_DIM)

    partial = jnp.dot(ctx.astype(jnp.bfloat16), Wo.astype(jnp.bfloat16),
                      preferred_element_type=jnp.float32)

    return _ring_allreduce(partial)[None]


# device time: 76189 ns/iter; 2.5804x vs baseline; 2.5804x over previous
import jax
import jax.numpy as jnp
from jax import lax
from jax.experimental import pallas as pl
from jax.experimental.pallas import tpu as pltpu

N_DEV = 4
WINDOW = 128
HEAD_DIM = 128
N_HEADS_LOC = 8
SCALE = 0.08838834764831843

SQ = 1024
D = 1024
HALF = SQ // 2
QTR = SQ // 4
KWIN = HALF + 2 * WINDOW


def _compute_half(off, kstart, x_ref, wq_ref, k_ref, v_ref, wo_ref):
    xh = x_ref[pl.ds(off, HALF), :]
    q = jnp.dot(xh, wq_ref[...], preferred_element_type=jnp.float32)
    q = q.astype(jnp.bfloat16).reshape(HALF, N_HEADS_LOC, HEAD_DIM)

    qi = lax.broadcasted_iota(jnp.int32, (HALF, KWIN), 0) + off
    ki = lax.broadcasted_iota(jnp.int32, (HALF, KWIN), 1) + kstart
    neg = jnp.float32(-1e9)
    mask = jnp.abs(qi - ki) <= WINDOW

    acc = jnp.zeros((HALF, D), jnp.float32)
    for h in range(N_HEADS_LOC):
        kh = k_ref[pl.ds(kstart, KWIN), h, :]
        vh = v_ref[pl.ds(kstart, KWIN), h, :]
        s = lax.dot_general(
            q[:, h, :], kh, (((1,), (1,)), ((), ())),
            preferred_element_type=jnp.float32,
        ) * SCALE
        s = jnp.where(mask, s, neg)
        m = jnp.max(s, axis=1, keepdims=True)
        w = jnp.exp(s - m)
        w = w / jnp.sum(w, axis=1, keepdims=True)
        ctx = lax.dot_general(
            w.astype(jnp.bfloat16), vh, (((1,), (0,)), ((), ())),
            preferred_element_type=jnp.float32,
        )
        acc = acc + lax.dot_general(
            ctx.astype(jnp.bfloat16),
            wo_ref[pl.ds(h * HEAD_DIM, HEAD_DIM), :],
            (((1,), (0,)), ((), ())),
            preferred_element_type=jnp.float32,
        )
    return acc


def _body(x_ref, wq_ref, k_ref, v_ref, wo_ref, out_ref,
          acc_ref, sendA, recv_a, sbuf, recv_b, qsend, recv_b2, hbuf,
          recv_a2, send_sems, recv_sems):
    my = lax.axis_index("i")
    pa = 3 - my
    pb = my ^ 1
    ha = my // 2
    my_h_off = ha * HALF
    ot_h_off = (1 - ha) * HALF
    my_q_off = my * QTR
    pb_q_off = pb * QTR
    qoff = my_q_off - my_h_off
    qp_loc = QTR - qoff
    kstart_other = jnp.where(ot_h_off == 0, 0, SQ - KWIN)
    kstart_mine = jnp.where(my_h_off == 0, 0, SQ - KWIN)

    barrier_sem = pltpu.get_barrier_semaphore()
    for nbr in (pa, pb):
        pl.semaphore_signal(
            barrier_sem, inc=1,
            device_id=(nbr,), device_id_type=pl.DeviceIdType.MESH,
        )

    p1 = _compute_half(ot_h_off, kstart_other, x_ref, wq_ref, k_ref, v_ref,
                       wo_ref)
    sendA[...] = p1.astype(jnp.bfloat16)
    pl.semaphore_wait(barrier_sem, 2)
    rdma_a = pltpu.make_async_remote_copy(
        src_ref=sendA, dst_ref=recv_a,
        send_sem=send_sems.at[0], recv_sem=recv_sems.at[0],
        device_id=(pa,), device_id_type=pl.DeviceIdType.MESH,
    )
    rdma_a.start()

    p2 = _compute_half(my_h_off, kstart_mine, x_ref, wq_ref, k_ref, v_ref,
                       wo_ref)
    rdma_a.wait()
    acc_ref[...] = p2 + recv_a[...].astype(jnp.float32)

    sbuf[...] = acc_ref[pl.ds(qp_loc, QTR), :].astype(jnp.bfloat16)
    rdma_b = pltpu.make_async_remote_copy(
        src_ref=sbuf, dst_ref=recv_b,
        send_sem=send_sems.at[1], recv_sem=recv_sems.at[1],
        device_id=(pb,), device_id_type=pl.DeviceIdType.MESH,
    )
    rdma_b.start()
    rdma_b.wait()
    myq = acc_ref[pl.ds(qoff, QTR), :] \
        + recv_b[...].astype(jnp.float32)

    qsend[...] = myq.astype(jnp.bfloat16)
    rdma_b2 = pltpu.make_async_remote_copy(
        src_ref=qsend, dst_ref=recv_b2,
        send_sem=send_sems.at[2], recv_sem=recv_sems.at[2],
        device_id=(pb,), device_id_type=pl.DeviceIdType.MESH,
    )
    rdma_b2.start()
    out_ref[pl.ds(my_q_off, QTR), :] = myq
    hbuf[pl.ds(qoff, QTR), :] = qsend[...]
    rdma_b2.wait()
    hbuf[pl.ds(qp_loc, QTR), :] = recv_b2[...]
    out_ref[pl.ds(pb_q_off, QTR), :] = recv_b2[...].astype(jnp.float32)

    rdma_a2 = pltpu.make_async_remote_copy(
        src_ref=hbuf, dst_ref=recv_a2,
        send_sem=send_sems.at[3], recv_sem=recv_sems.at[3],
        device_id=(pa,), device_id_type=pl.DeviceIdType.MESH,
    )
    rdma_a2.start()
    rdma_a2.wait()
    out_ref[pl.ds(ot_h_off, HALF), :] = recv_a2[...].astype(jnp.float32)


def kernel(x, Wq, K_ext, V_ext, Wo):
    my = lax.axis_index("i")
    xb = x[0].astype(jnp.bfloat16)
    wq = Wq.astype(jnp.bfloat16)
    K = lax.dynamic_slice_in_dim(K_ext[0], my * N_HEADS_LOC, N_HEADS_LOC,
                                 axis=1).astype(jnp.bfloat16)
    V = lax.dynamic_slice_in_dim(V_ext[0], my * N_HEADS_LOC, N_HEADS_LOC,
                                 axis=1).astype(jnp.bfloat16)
    wo = Wo.astype(jnp.bfloat16)

    out = pl.pallas_call(
        _body,
        out_shape=jax.ShapeDtypeStruct((SQ, D), jnp.float32),
        in_specs=[pl.BlockSpec(memory_space=pltpu.VMEM)] * 5,
        out_specs=pl.BlockSpec(memory_space=pltpu.VMEM),
        scratch_shapes=[
            pltpu.VMEM((HALF, D), jnp.float32),
            pltpu.VMEM((HALF, D), jnp.bfloat16),
            pltpu.VMEM((HALF, D), jnp.bfloat16),
            pltpu.VMEM((QTR, D), jnp.bfloat16),
            pltpu.VMEM((QTR, D), jnp.bfloat16),
            pltpu.VMEM((QTR, D), jnp.bfloat16),
            pltpu.VMEM((QTR, D), jnp.bfloat16),
            pltpu.VMEM((HALF, D), jnp.bfloat16),
            pltpu.VMEM((HALF, D), jnp.bfloat16),
            pltpu.SemaphoreType.DMA((4,)),
            pltpu.SemaphoreType.DMA((4,)),
        ],
        compiler_params=pltpu.CompilerParams(collective_id=0),
    )(xb, wq, K, V, wo)
    return out[None]


# device time: 74745 ns/iter; 2.6302x vs baseline; 1.0193x over previous
import jax
import jax.numpy as jnp
from jax import lax
from jax.experimental import pallas as pl
from jax.experimental.pallas import tpu as pltpu

N_DEV = 4
WINDOW = 128
HEAD_DIM = 128
N_HEADS_LOC = 8
SCALE = 0.08838834764831843

SQ = 1024
D = 1024
HALF = SQ // 2
QTR = SQ // 4
KWIN = QTR + 2 * WINDOW


def _dot(a, b, dims):
    return lax.dot_general(a, b, (dims, ((), ())),
                           preferred_element_type=jnp.float32)


def _compute_quarter(qidx, xb_ref, wqb_ref, k_ref, v_ref, wob_ref):
    off = qidx * QTR
    kstart = jnp.clip(off - WINDOW, 0, SQ - KWIN)

    xh = xb_ref[pl.ds(off, QTR), :]
    q = jnp.dot(xh, wqb_ref[...], preferred_element_type=jnp.float32)
    q = q.astype(jnp.bfloat16).reshape(QTR, N_HEADS_LOC, HEAD_DIM)

    qi = lax.broadcasted_iota(jnp.int32, (QTR, KWIN), 0) + off
    ki = lax.broadcasted_iota(jnp.int32, (QTR, KWIN), 1) + kstart
    mask = jnp.abs(qi - ki) <= WINDOW
    neg = jnp.float32(-1e9)

    acc = jnp.zeros((QTR, D), jnp.float32)
    for h in range(N_HEADS_LOC):
        kh = k_ref[pl.ds(kstart, KWIN), h, :].astype(jnp.bfloat16)
        vh = v_ref[pl.ds(kstart, KWIN), h, :].astype(jnp.bfloat16)
        s = _dot(q[:, h, :], kh, ((1,), (1,))) * SCALE
        s = jnp.where(mask, s, neg)
        m = jnp.max(s, axis=1, keepdims=True)
        w = jnp.exp(s - m)
        w = w / jnp.sum(w, axis=1, keepdims=True)
        ctx = _dot(w.astype(jnp.bfloat16), vh, ((1,), (0,)))
        acc = acc + _dot(ctx.astype(jnp.bfloat16),
                         wob_ref[pl.ds(h * HEAD_DIM, HEAD_DIM), :],
                         ((1,), (0,)))
    return acc


def _body(x_ref, wq_ref, k_ref, v_ref, wo_ref, out_ref,
          xb_ref, wqb_ref, wob_ref,
          sendA, recv_a, sbuf, recv_b, qsend, recv_b2, hbuf, recv_a2,
          send_sems, recv_sems):
    my = lax.axis_index("i")
    pa = 3 - my
    pb = my ^ 1
    ha = my // 2
    my_h_off = ha * HALF
    ot_h_off = (1 - ha) * HALF
    my_q_off = my * QTR
    pb_q_off = pb * QTR
    qoff = my_q_off - my_h_off
    qoffp = QTR - qoff
    q_first = 2 * (1 - ha)

    barrier_sem = pltpu.get_barrier_semaphore()
    for nbr in (pa, pb):
        pl.semaphore_signal(
            barrier_sem, inc=1,
            device_id=(nbr,), device_id_type=pl.DeviceIdType.MESH,
        )

    xb_ref[...] = x_ref[...].astype(jnp.bfloat16)
    wqb_ref[...] = wq_ref[...].astype(jnp.bfloat16)
    wob_ref[...] = wo_ref[...].astype(jnp.bfloat16)

    def rc(src, dst, sem_idx, dev):
        return pltpu.make_async_remote_copy(
            src_ref=src, dst_ref=dst,
            send_sem=send_sems.at[sem_idx], recv_sem=recv_sems.at[sem_idx],
            device_id=(dev,), device_id_type=pl.DeviceIdType.MESH,
        )

    p = _compute_quarter(q_first, xb_ref, wqb_ref, k_ref, v_ref, wob_ref)
    sendA[pl.ds(0, QTR), :] = p.astype(jnp.bfloat16)
    pl.semaphore_wait(barrier_sem, 2)
    rdma_a1 = rc(sendA.at[pl.ds(0, QTR)], recv_a.at[pl.ds(0, QTR)], 0, pa)
    rdma_a1.start()

    p = _compute_quarter(q_first + 1, xb_ref, wqb_ref, k_ref, v_ref, wob_ref)
    sendA[pl.ds(QTR, QTR), :] = p.astype(jnp.bfloat16)
    rdma_a2 = rc(sendA.at[pl.ds(QTR, QTR)], recv_a.at[pl.ds(QTR, QTR)], 1, pa)
    rdma_a2.start()

    p2b = _compute_quarter(pb, xb_ref, wqb_ref, k_ref, v_ref, wob_ref)
    rdma_a1.wait()
    rdma_a2.wait()
    sbuf[...] = (p2b + recv_a[pl.ds(qoffp, QTR), :].astype(jnp.float32)
                 ).astype(jnp.bfloat16)
    rdma_b = rc(sbuf, recv_b, 2, pb)
    rdma_b.start()

    p2a = _compute_quarter(my, xb_ref, wqb_ref, k_ref, v_ref, wob_ref)
    rdma_b.wait()
    myq = (p2a + recv_a[pl.ds(qoff, QTR), :].astype(jnp.float32)
           + recv_b[...].astype(jnp.float32))

    qsend[...] = myq.astype(jnp.bfloat16)
    rdma_b2 = rc(qsend, recv_b2, 3, pb)
    rdma_b2.start()
    out_ref[pl.ds(my_q_off, QTR), :] = myq
    hbuf[pl.ds(qoff, QTR), :] = qsend[...]
    rdma_b2.wait()
    hbuf[pl.ds(qoffp, QTR), :] = recv_b2[...]
    out_ref[pl.ds(pb_q_off, QTR), :] = recv_b2[...].astype(jnp.float32)

    rdma_a3 = rc(hbuf, recv_a2, 4, pa)
    rdma_a3.start()
    rdma_a3.wait()
    out_ref[pl.ds(ot_h_off, HALF), :] = recv_a2[...].astype(jnp.float32)


def kernel(x, Wq, K_ext, V_ext, Wo):
    my = lax.axis_index("i")
    K = lax.dynamic_slice_in_dim(K_ext[0], my * N_HEADS_LOC, N_HEADS_LOC,
                                 axis=1)
    V = lax.dynamic_slice_in_dim(V_ext[0], my * N_HEADS_LOC, N_HEADS_LOC,
                                 axis=1)

    out = pl.pallas_call(
        _body,
        out_shape=jax.ShapeDtypeStruct((SQ, D), jnp.float32),
        in_specs=[pl.BlockSpec(memory_space=pltpu.VMEM)] * 5,
        out_specs=pl.BlockSpec(memory_space=pltpu.VMEM),
        scratch_shapes=[
            pltpu.VMEM((SQ, D), jnp.bfloat16),
            pltpu.VMEM((D, D), jnp.bfloat16),
            pltpu.VMEM((D, D), jnp.bfloat16),
            pltpu.VMEM((HALF, D), jnp.bfloat16),
            pltpu.VMEM((HALF, D), jnp.bfloat16),
            pltpu.VMEM((QTR, D), jnp.bfloat16),
            pltpu.VMEM((QTR, D), jnp.bfloat16),
            pltpu.VMEM((QTR, D), jnp.bfloat16),
            pltpu.VMEM((QTR, D), jnp.bfloat16),
            pltpu.VMEM((HALF, D), jnp.bfloat16),
            pltpu.VMEM((HALF, D), jnp.bfloat16),
            pltpu.SemaphoreType.DMA((5,)),
            pltpu.SemaphoreType.DMA((5,)),
        ],
        compiler_params=pltpu.CompilerParams(collective_id=0),
    )(x[0], Wq, K, V, Wo)
    return out[None]


# device time: 70490 ns/iter; 2.7890x vs baseline; 1.0604x over previous
import jax
import jax.numpy as jnp
from jax import lax
from jax.experimental import pallas as pl
from jax.experimental.pallas import tpu as pltpu

N_DEV = 4
WINDOW = 128
HEAD_DIM = 128
N_HEADS_LOC = 8
SCALE = 0.08838834764831843

SQ = 1024
D = 1024
HALF = SQ // 2
QTR = SQ // 4
KWIN = QTR + 2 * WINDOW


def _dot(a, b, dims):
    return lax.dot_general(a, b, (dims, ((), ())),
                           preferred_element_type=jnp.float32)


def _compute_quarter(qidx, xb_ref, wqb_ref, kb_ref, vb_ref, wob_ref):
    off = qidx * QTR
    kstart = pl.multiple_of(jnp.clip(off - WINDOW, 0, SQ - KWIN), WINDOW)

    xh = xb_ref[pl.ds(off, QTR), :]
    q = jnp.dot(xh, wqb_ref[...], preferred_element_type=jnp.float32)
    q = q.astype(jnp.bfloat16)

    qi = lax.broadcasted_iota(jnp.int32, (QTR, KWIN), 0) + off
    ki = lax.broadcasted_iota(jnp.int32, (QTR, KWIN), 1) + kstart
    mask = jnp.abs(qi - ki) <= WINDOW

    acc = jnp.zeros((QTR, D), jnp.float32)
    for h in range(N_HEADS_LOC):
        kh = kb_ref[h, pl.ds(kstart, KWIN), :]
        vh = vb_ref[h, pl.ds(kstart, KWIN), :]
        s = _dot(q[:, h * HEAD_DIM:(h + 1) * HEAD_DIM], kh,
                 ((1,), (1,))) * SCALE
        w = jnp.exp(jnp.where(mask, s, jnp.float32(-1e9)))
        w = w / jnp.sum(w, axis=1, keepdims=True)
        ctx = _dot(w.astype(jnp.bfloat16), vh, ((1,), (0,)))
        acc = acc + _dot(ctx.astype(jnp.bfloat16),
                         wob_ref[pl.ds(h * HEAD_DIM, HEAD_DIM), :],
                         ((1,), (0,)))
    return acc


def _body(x_ref, wq_ref, k_any, v_any, wo_ref, out_ref,
          xb_ref, wqb_ref, wob_ref, kslab, vslab, kb_ref, vb_ref,
          sendA, recv_a, sbuf, recv_b, qsend, recv_b2, recv_a2,
          copy_sems, send_sems, recv_sems):
    my = lax.axis_index("i")
    pa = 3 - my
    pb = my ^ 1
    ha = my // 2
    my_h_off = ha * HALF
    ot_h_off = (1 - ha) * HALF
    my_q_off = my * QTR
    pb_q_off = pb * QTR
    qoff = my_q_off - my_h_off
    qoffp = QTR - qoff
    q_first = 2 * (1 - ha)
    hb = my * N_HEADS_LOC

    kcopy = pltpu.make_async_copy(
        k_any.at[0, :, pl.ds(hb, N_HEADS_LOC), :], kslab, copy_sems.at[0])
    vcopy = pltpu.make_async_copy(
        v_any.at[0, :, pl.ds(hb, N_HEADS_LOC), :], vslab, copy_sems.at[1])
    kcopy.start()
    vcopy.start()

    barrier_sem = pltpu.get_barrier_semaphore()
    for nbr in (pa, pb):
        pl.semaphore_signal(
            barrier_sem, inc=1,
            device_id=(nbr,), device_id_type=pl.DeviceIdType.MESH,
        )

    xb_ref[...] = x_ref[...].astype(jnp.bfloat16)
    wqb_ref[...] = wq_ref[...].astype(jnp.bfloat16)
    wob_ref[...] = wo_ref[...].astype(jnp.bfloat16)
    kcopy.wait()
    vcopy.wait()
    for h in range(N_HEADS_LOC):
        kb_ref[h, :, :] = kslab[:, h, :].astype(jnp.bfloat16)
        vb_ref[h, :, :] = vslab[:, h, :].astype(jnp.bfloat16)

    def rc(src, dst, sem_idx, dev):
        return pltpu.make_async_remote_copy(
            src_ref=src, dst_ref=dst,
            send_sem=send_sems.at[sem_idx], recv_sem=recv_sems.at[sem_idx],
            device_id=(dev,), device_id_type=pl.DeviceIdType.MESH,
        )

    p = _compute_quarter(q_first, xb_ref, wqb_ref, kb_ref, vb_ref, wob_ref)
    sendA[pl.ds(0, QTR), :] = p.astype(jnp.bfloat16)
    pl.semaphore_wait(barrier_sem, 2)
    rdma_a1 = rc(sendA.at[pl.ds(0, QTR)], recv_a.at[pl.ds(0, QTR)], 0, pa)
    rdma_a1.start()

    p = _compute_quarter(q_first + 1, xb_ref, wqb_ref, kb_ref, vb_ref,
                         wob_ref)
    sendA[pl.ds(QTR, QTR), :] = p.astype(jnp.bfloat16)
    rdma_a2 = rc(sendA.at[pl.ds(QTR, QTR)], recv_a.at[pl.ds(QTR, QTR)], 1, pa)
    rdma_a2.start()

    p2b = _compute_quarter(pb, xb_ref, wqb_ref, kb_ref, vb_ref, wob_ref)
    rdma_a1.wait()
    rdma_a2.wait()
    sbuf[...] = (p2b + recv_a[pl.ds(qoffp, QTR), :].astype(jnp.float32)
                 ).astype(jnp.bfloat16)
    rdma_b = rc(sbuf, recv_b, 2, pb)
    rdma_b.start()

    p2a = _compute_quarter(my, xb_ref, wqb_ref, kb_ref, vb_ref, wob_ref)
    rdma_b.wait()
    myq = (p2a + recv_a[pl.ds(qoff, QTR), :].astype(jnp.float32)
           + recv_b[...].astype(jnp.float32))

    qsend[...] = myq.astype(jnp.bfloat16)
    rdma_b2 = rc(qsend, recv_b2, 3, pb)
    rdma_b2.start()
    rdma_a3a = rc(qsend, recv_a2.at[pl.ds(qoff, QTR)], 4, pa)
    rdma_a3a.start()
    out_ref[pl.ds(my_q_off, QTR), :] = myq
    rdma_b2.wait()
    rdma_a3b = rc(recv_b2, recv_a2.at[pl.ds(qoffp, QTR)], 5, pa)
    rdma_a3b.start()
    out_ref[pl.ds(pb_q_off, QTR), :] = recv_b2[...].astype(jnp.float32)
    rdma_a3a.wait()
    rdma_a3b.wait()
    out_ref[pl.ds(ot_h_off, HALF), :] = recv_a2[...].astype(jnp.float32)


def kernel(x, Wq, K_ext, V_ext, Wo):
    out = pl.pallas_call(
        _body,
        out_shape=jax.ShapeDtypeStruct((SQ, D), jnp.float32),
        in_specs=[
            pl.BlockSpec(memory_space=pltpu.VMEM),
            pl.BlockSpec(memory_space=pltpu.VMEM),
            pl.BlockSpec(memory_space=pltpu.MemorySpace.HBM),
            pl.BlockSpec(memory_space=pltpu.MemorySpace.HBM),
            pl.BlockSpec(memory_space=pltpu.VMEM),
        ],
        out_specs=pl.BlockSpec(memory_space=pltpu.VMEM),
        scratch_shapes=[
            pltpu.VMEM((SQ, D), jnp.bfloat16),
            pltpu.VMEM((D, D), jnp.bfloat16),
            pltpu.VMEM((D, D), jnp.bfloat16),
            pltpu.VMEM((SQ, N_HEADS_LOC, HEAD_DIM), jnp.float32),
            pltpu.VMEM((SQ, N_HEADS_LOC, HEAD_DIM), jnp.float32),
            pltpu.VMEM((N_HEADS_LOC, SQ, HEAD_DIM), jnp.bfloat16),
            pltpu.VMEM((N_HEADS_LOC, SQ, HEAD_DIM), jnp.bfloat16),
            pltpu.VMEM((HALF, D), jnp.bfloat16),
            pltpu.VMEM((HALF, D), jnp.bfloat16),
            pltpu.VMEM((QTR, D), jnp.bfloat16),
            pltpu.VMEM((QTR, D), jnp.bfloat16),
            pltpu.VMEM((QTR, D), jnp.bfloat16),
            pltpu.VMEM((QTR, D), jnp.bfloat16),
            pltpu.VMEM((HALF, D), jnp.bfloat16),
            pltpu.SemaphoreType.DMA((2,)),
            pltpu.SemaphoreType.DMA((6,)),
            pltpu.SemaphoreType.DMA((6,)),
        ],
        compiler_params=pltpu.CompilerParams(collective_id=0),
    )(x[0], Wq, K_ext, V_ext, Wo)
    return out[None]
